# baseline (speedup 1.0000x reference)
import jax
import jax.numpy as jnp
from jax import lax
from jax.experimental import pallas as pl
from jax.experimental.pallas import tpu as pltpu
from jax.experimental.pallas import tpu_sc as plsc

NUM_MODELS = 8
NUM_BOXES = 100000
DIM = 64
BATCH = 4096

_INFO = plsc.get_sparse_core_info()
NC = _INFO.num_cores
NS = _INFO.num_subcores
LANES = _INFO.num_lanes
NW = NC * NS
NROWS = NUM_MODELS * DIM
RPW = NROWS // NW

HA = 49920
HB = 50048
TAIL = 128
TS0 = NUM_BOXES - TAIL
NVREG = BATCH // LANES


def _sc_body(z_hbm, ld_hbm, zt_hbm, ldt_hbm, idx_hbm, out_hbm,
             idx_v, buf_v, zg_v, cap_v, sem_a, sem_b, sem_t):
    cid = lax.axis_index("c")
    sid = lax.axis_index("s")
    wid = sid * NC + cid
    r0 = wid * RPW

    pltpu.sync_copy(idx_hbm, idx_v)

    def copy_a(tab, r):
        return pltpu.make_async_copy(
            tab.at[r, pl.ds(0, HA)], buf_v.at[pl.ds(0, HA)], sem_a)

    def copy_b(tab, r):
        return pltpu.make_async_copy(
            tab.at[r, pl.ds(HA, HB)], buf_v.at[pl.ds(HA, HB)], sem_b)

    def copy_t(tail, r):
        return pltpu.make_async_copy(
            tail.at[pl.ds(r * TAIL, TAIL)], buf_v.at[pl.ds(TS0, TAIL)], sem_t)

    def start_b(tab, tail, r):
        copy_b(tab, r).start()
        copy_t(tail, r).start()

    copy_a(z_hbm, r0).start()

    def do_row(i, carry):
        r = r0 + i
        rn = jnp.minimum(r + 1, r0 + RPW - 1)
        m = r // DIM
        d = r % DIM

        start_b(z_hbm, zt_hbm, r)
        copy_a(z_hbm, r).wait()

        def pass_za(j, c):
            sl = pl.ds(j * LANES, LANES)
            g = plsc.load_gather(buf_v, [jnp.minimum(idx_v[sl], HA - 1)])
            zg_v[sl] = g
            return c

        lax.fori_loop(0, NVREG, pass_za, 0, unroll=4)

        copy_a(ld_hbm, r).start()
        copy_b(z_hbm, r).wait()
        copy_t(zt_hbm, r).wait()

        def pass_zb(j, c):
            sl = pl.ds(j * LANES, LANES)
            ii = idx_v[sl]
            g = plsc.load_gather(buf_v, [jnp.maximum(ii, HA)])
            zg_v[sl] = jnp.where(ii >= HA, g, zg_v[sl])
            return c

        lax.fori_loop(0, NVREG, pass_zb, 0, unroll=4)

        start_b(ld_hbm, ldt_hbm, r)
        copy_a(ld_hbm, r).wait()

        def pass_la(j, c):
            sl = pl.ds(j * LANES, LANES)
            g = plsc.load_gather(buf_v, [jnp.minimum(idx_v[sl], HA - 1)])
            cap_v[sl] = zg_v[sl] + jnp.exp(g)
            return c

        lax.fori_loop(0, NVREG, pass_la, 0, unroll=4)

        copy_a(z_hbm, rn).start()
        copy_b(ld_hbm, r).wait()
        copy_t(ldt_hbm, r).wait()

        def pass_lb(j, c):
            sl = pl.ds(j * LANES, LANES)
            ii = idx_v[sl]
            g = plsc.load_gather(buf_v, [jnp.maximum(ii, HA)])
            cap_v[sl] = jnp.where(ii >= HA, zg_v[sl] + jnp.exp(g), cap_v[sl])
            return c

        lax.fori_loop(0, NVREG, pass_lb, 0, unroll=4)

        pltpu.sync_copy(zg_v, out_hbm.at[m, 0, d])
        pltpu.sync_copy(cap_v, out_hbm.at[m, 1, d])
        return carry

    lax.fori_loop(0, RPW, do_row, 0)

    copy_a(z_hbm, r0 + RPW - 1).wait()


@jax.jit
def kernel(box_indices, z, logdelta):
    zT = z.transpose(0, 2, 1).reshape(NROWS, NUM_BOXES)
    ldT = logdelta.transpose(0, 2, 1).reshape(NROWS, NUM_BOXES)
    zt = lax.slice(zT, (0, TS0), (NROWS, NUM_BOXES)).reshape(NROWS * TAIL)
    ldt = lax.slice(ldT, (0, TS0), (NROWS, NUM_BOXES)).reshape(NROWS * TAIL)
    idx = box_indices.astype(jnp.int32)

    mesh = plsc.VectorSubcoreMesh(core_axis_name="c", subcore_axis_name="s")
    out = pl.kernel(
        _sc_body,
        out_type=jax.ShapeDtypeStruct((NUM_MODELS, 2, DIM, BATCH),
                                      jnp.float32),
        mesh=mesh,
        compiler_params=pltpu.CompilerParams(needs_layout_passes=False),
        scratch_types=[
            pltpu.VMEM((BATCH,), jnp.int32),
            pltpu.VMEM((NUM_BOXES,), jnp.float32),
            pltpu.VMEM((BATCH,), jnp.float32),
            pltpu.VMEM((BATCH,), jnp.float32),
            pltpu.SemaphoreType.DMA,
            pltpu.SemaphoreType.DMA,
            pltpu.SemaphoreType.DMA,
        ],
    )(zT, ldT, zt, ldt, idx)

    return out.transpose(0, 3, 1, 2)


# speedup vs baseline: 1.0382x; 1.0382x over previous
import jax
import jax.numpy as jnp
from jax import lax
from jax.experimental import pallas as pl
from jax.experimental.pallas import tpu as pltpu
from jax.experimental.pallas import tpu_sc as plsc

NUM_MODELS = 8
NUM_BOXES = 100000
DIM = 64
BATCH = 4096

_INFO = plsc.get_sparse_core_info()
NC = _INFO.num_cores
NS = _INFO.num_subcores
LANES = _INFO.num_lanes
NW = NC * NS
NROWS = NUM_MODELS * DIM
RPW = NROWS // NW
NVREG = BATCH // LANES


def _sc_body(z_hbm, ld_hbm, idx_hbm, out_hbm,
             idx_v, buf_v, zg_v, cap_v, sem_f, sem_oz, sem_oc):
    cid = lax.axis_index("c")
    sid = lax.axis_index("s")
    wid = sid * NC + cid
    r0 = wid * RPW
    m0 = r0 // DIM
    d0 = r0 % DIM

    pltpu.sync_copy(idx_hbm, idx_v)

    def fill(tab, r):
        return pltpu.make_async_copy(tab.at[r], buf_v, sem_f)

    def out_z(r, src):
        return pltpu.make_async_copy(src, out_hbm.at[r // DIM, 0, r % DIM],
                                     sem_oz)

    def out_c(r, src):
        return pltpu.make_async_copy(src, out_hbm.at[r // DIM, 1, r % DIM],
                                     sem_oc)

    fill(z_hbm, r0).start()
    out_z(r0, zg_v).start()
    out_c(r0, cap_v).start()

    def do_row(i, carry):
        r = r0 + i
        rn = jnp.minimum(r + 1, r0 + RPW - 1)

        fill(z_hbm, r).wait()
        out_z(r, zg_v).wait()

        def zgather(j, c):
            sl = pl.ds(j * LANES, LANES)
            zg_v[sl] = plsc.load_gather(buf_v, [idx_v[sl]])
            return c

        lax.fori_loop(0, NVREG, zgather, 0, unroll=4)

        fill(ld_hbm, r).start()
        out_z(r, zg_v).start()
        fill(ld_hbm, r).wait()
        out_c(r, cap_v).wait()

        def lgather(j, c):
            sl = pl.ds(j * LANES, LANES)
            cap_v[sl] = zg_v[sl] + jnp.exp(
                plsc.load_gather(buf_v, [idx_v[sl]]))
            return c

        lax.fori_loop(0, NVREG, lgather, 0, unroll=4)

        fill(z_hbm, rn).start()
        out_c(r, cap_v).start()
        return carry

    lax.fori_loop(0, RPW, do_row, 0)

    fill(z_hbm, r0 + RPW - 1).wait()
    out_z(r0, zg_v).wait()
    out_c(r0, cap_v).wait()


@jax.jit
def kernel(box_indices, z, logdelta):
    zT = z.transpose(0, 2, 1).reshape(NROWS, NUM_BOXES)
    ldT = logdelta.transpose(0, 2, 1).reshape(NROWS, NUM_BOXES)
    idx = box_indices.astype(jnp.int32)

    mesh = plsc.VectorSubcoreMesh(core_axis_name="c", subcore_axis_name="s")
    out = pl.kernel(
        _sc_body,
        out_type=jax.ShapeDtypeStruct((NUM_MODELS, 2, DIM, BATCH),
                                      jnp.float32),
        mesh=mesh,
        compiler_params=pltpu.CompilerParams(needs_layout_passes=False),
        scratch_types=[
            pltpu.VMEM((BATCH,), jnp.int32),
            pltpu.VMEM((NUM_BOXES,), jnp.float32),
            pltpu.VMEM((BATCH,), jnp.float32),
            pltpu.VMEM((BATCH,), jnp.float32),
            pltpu.SemaphoreType.DMA,
            pltpu.SemaphoreType.DMA,
            pltpu.SemaphoreType.DMA,
        ],
    )(zT, ldT, idx)

    return out.transpose(0, 3, 1, 2)
